# baseline (speedup 1.0000x reference)
import functools

import jax
import jax.numpy as jnp
from jax import lax
from jax.experimental import pallas as pl
from jax.experimental.pallas import tpu as pltpu
from jax.experimental.pallas import tpu_sc as plsc

_VOCAB = 100000
_SEQ = 200
_D = 128
_BATCH = 4096
_NC = 2
_NS = 16
_NW = _NC * _NS
_ROWS = _BATCH * _SEQ
_RPW = _ROWS // _NW
_CHUNK = _SEQ
_NCHUNK = _RPW // _CHUNK
_NBUF = 4
_NIDX = 4
_IDXPAD = 384
_LANES = 16
_NSLICE = _D // _LANES
_GROUP = 8
_SCALE = float(_D) ** 0.5


def _emb_body(idx_hbm, tok_hbm, pos_hbm, out_hbm, idx_v, pos_v, rows_v,
              g0, g1, g2, g3, s0, s1, s2, s3, i0, i1, i2, i3):
    wid = lax.axis_index("s") * _NC + lax.axis_index("c")
    base = wid * _RPW
    gsems = (g0, g1, g2, g3)
    ssems = (s0, s1, s2, s3)
    isems = (i0, i1, i2, i3)

    def idx_off(i):
        return lax.rem(i * _CHUNK, 128)

    def idx_desc(i, q):
        slab = i * _CHUNK - idx_off(i)
        return pltpu.make_async_copy(
            idx_hbm.at[pl.ds(base + slab, _IDXPAD)],
            idx_v.at[pl.ds(q * _IDXPAD, _IDXPAD)], isems[q])

    def gather_desc(i, k, q):
        return pltpu.make_async_copy(
            tok_hbm.at[idx_v.at[pl.ds(q * _IDXPAD + idx_off(i), _CHUNK)]],
            rows_v.at[k], gsems[k])

    def scatter_desc(i, k):
        return pltpu.make_async_copy(
            rows_v.at[k], out_hbm.at[pl.ds(base + i * _CHUNK, _CHUNK)],
            ssems[k])

    def compute(i, k, q):
        o0 = idx_off(i)

        def group_step(g, c2):
            idxv = idx_v[pl.ds(q * _IDXPAD + o0 + g * _GROUP, _LANES)]
            af = jnp.where(idxv != 0, _SCALE, 0.0).astype(jnp.float32)
            bf = jnp.where(idxv != 0, 1.0, 0.0).astype(jnp.float32)
            for kk in range(_GROUP):
                r = g * _GROUP + kk
                a = af[kk]
                b = bf[kk]
                for j in range(_NSLICE):
                    sl = pl.ds(j * _LANES, _LANES)
                    rows_v[k, r, sl] = rows_v[k, r, sl] * a + pos_v[r, sl] * b
            return c2

        lax.fori_loop(0, _CHUNK // _GROUP, group_step, 0, unroll=1)

    def iteration(i, k, q, first, steady):
        gather_desc(i, k, q).wait()
        compute(i, k, q)
        scatter_desc(i, k).start()
        if not first:
            scatter_desc(i - 2, (k + 2) % _NBUF).wait()
        q2 = (q + 2) % _NIDX
        k2 = (k + 2) % _NBUF
        if steady:

            @pl.when(i + 2 < _NCHUNK)
            def _():
                idx_desc(i + 2, q2).wait()
                gather_desc(i + 2, k2, q2).start()

            @pl.when(i + _NIDX < _NCHUNK)
            def _():
                idx_desc(i + _NIDX, q).start()
        else:
            idx_desc(i + 2, q2).wait()
            gather_desc(i + 2, k2, q2).start()
            idx_desc(i + _NIDX, q).start()

    pltpu.sync_copy(pos_hbm, pos_v)
    for i in range(_NIDX):
        idx_desc(i, i).start()
    idx_desc(0, 0).wait()
    gather_desc(0, 0, 0).start()
    idx_desc(1, 1).wait()
    gather_desc(1, 1, 1).start()
    iteration(0, 0, 0, first=True, steady=False)
    iteration(1, 1, 1, first=True, steady=False)
    iteration(2, 2, 2, first=False, steady=False)
    iteration(3, 3, 3, first=False, steady=False)

    def outer(g, carry):
        for u in range(4):
            i = g * 4 + u + 4
            iteration(i, u, u, first=False, steady=True)
        return carry

    lax.fori_loop(0, (_NCHUNK - 4) // 4, outer, 0, unroll=1)
    scatter_desc(_NCHUNK - 2, (_NCHUNK - 2) % _NBUF).wait()
    scatter_desc(_NCHUNK - 1, (_NCHUNK - 1) % _NBUF).wait()


_emb = functools.partial(
    pl.kernel,
    out_type=jax.ShapeDtypeStruct((_ROWS, _D), jnp.float32),
    mesh=plsc.VectorSubcoreMesh(core_axis_name="c", subcore_axis_name="s"),
    scratch_types=[
        pltpu.VMEM((_NIDX * _IDXPAD,), jnp.int32),
        pltpu.VMEM((_SEQ, _D), jnp.float32),
        pltpu.VMEM((_NBUF, _CHUNK, _D), jnp.float32),
    ] + [pltpu.SemaphoreType.DMA] * (_NBUF + _NBUF + _NIDX),
)(_emb_body)


def kernel(inputs, token_table, pos_table):
    idx = jnp.pad(inputs.reshape(-1), (0, 128))
    out = _emb(idx, token_table, pos_table)
    return out.reshape(_BATCH, _SEQ, _D)
